# baseline (speedup 1.0000x reference)
import functools

import jax
import jax.numpy as jnp
from jax import lax
from jax.experimental import pallas as pl
from jax.experimental.pallas import tpu as pltpu

NU = 50000
NM = 10000
NE = 800000
NL = 100000
HID = 64
MF = 24
BLK = 1000


def _mi_body(x_ref, w_ref, b_ref, emb_ref, o_ref):
    o_ref[...] = (jnp.dot(x_ref[...], w_ref[...],
                          preferred_element_type=jnp.float32)
                  + b_ref[...] + emb_ref[...])


def _movie_init(movie_x, W, b, emb):
    n = movie_x.shape[0]
    return pl.pallas_call(
        _mi_body,
        grid=(n // BLK,),
        in_specs=[pl.BlockSpec((BLK, MF), lambda i: (i, 0)),
                  pl.BlockSpec((MF, HID), lambda i: (0, 0)),
                  pl.BlockSpec((1, HID), lambda i: (0, 0)),
                  pl.BlockSpec((BLK, HID), lambda i: (i, 0))],
        out_specs=pl.BlockSpec((BLK, HID), lambda i: (i, 0)),
        out_shape=jax.ShapeDtypeStruct((n, HID), jnp.float32),
    )(movie_x, W, b.reshape(1, HID), emb)


def _mm_body(x_ref, w_ref, o_ref):
    o_ref[...] = jnp.dot(x_ref[...], w_ref[...],
                         preferred_element_type=jnp.float32)


def _matmul(x, W):
    n, k = x.shape
    m = W.shape[1]
    return pl.pallas_call(
        _mm_body,
        grid=(n // BLK,),
        in_specs=[pl.BlockSpec((BLK, k), lambda i: (i, 0)),
                  pl.BlockSpec((k, m), lambda i: (0, 0))],
        out_specs=pl.BlockSpec((BLK, m), lambda i: (i, 0)),
        out_shape=jax.ShapeDtypeStruct((n, m), jnp.float32),
    )(x, W)


def _p1_body(agg_ref, deg_ref, x_ref, wl_ref, bl_ref, wr_ref,
             u_ref, stats_ref, acc):
    i = pl.program_id(0)
    agg = jnp.concatenate([agg_ref[0], agg_ref[1]], axis=1)
    mean = agg * (1.0 / jnp.maximum(deg_ref[...], 1.0))
    u = (jnp.dot(mean, wl_ref[...], preferred_element_type=jnp.float32)
         + bl_ref[...]
         + jnp.dot(x_ref[...], wr_ref[...],
                   preferred_element_type=jnp.float32))
    u_ref[...] = u

    @pl.when(i == 0)
    def _():
        acc[...] = jnp.zeros_like(acc)

    acc[0:1, :] += jnp.sum(u, axis=0, keepdims=True)
    acc[1:2, :] += jnp.sum(u * u, axis=0, keepdims=True)

    @pl.when(i == pl.num_programs(0) - 1)
    def _():
        n = BLK * pl.num_programs(0) * 1.0
        m = acc[0:1, :] / n
        v = acc[1:2, :] / n - m * m
        stats_ref[0:1, :] = m
        stats_ref[1:2, :] = 1.0 / jnp.sqrt(v + 1e-5)
        stats_ref[2:8, :] = jnp.zeros((6, HID), jnp.float32)


def _pass1(agg2, deg, x, Wl, bl, Wr):
    n = x.shape[0]
    return pl.pallas_call(
        _p1_body,
        grid=(n // BLK,),
        in_specs=[pl.BlockSpec((2, BLK, 32), lambda i: (0, i, 0)),
                  pl.BlockSpec((BLK, 1), lambda i: (i, 0)),
                  pl.BlockSpec((BLK, HID), lambda i: (i, 0)),
                  pl.BlockSpec((HID, HID), lambda i: (0, 0)),
                  pl.BlockSpec((1, HID), lambda i: (0, 0)),
                  pl.BlockSpec((HID, HID), lambda i: (0, 0))],
        out_specs=[pl.BlockSpec((BLK, HID), lambda i: (i, 0)),
                   pl.BlockSpec((8, HID), lambda i: (0, 0))],
        out_shape=[jax.ShapeDtypeStruct((n, HID), jnp.float32),
                   jax.ShapeDtypeStruct((8, HID), jnp.float32)],
        scratch_shapes=[pltpu.VMEM((8, HID), jnp.float32)],
    )(agg2, deg, x, Wl, bl.reshape(1, HID), Wr)


def _p2_body(u_ref, stats_ref, g_ref, b_ref, o_ref):
    o_ref[...] = jnp.maximum(
        (u_ref[...] - stats_ref[0:1, :]) * stats_ref[1:2, :] * g_ref[...]
        + b_ref[...], 0.0)


def _pass2(U, stats, g, b):
    n = U.shape[0]
    return pl.pallas_call(
        _p2_body,
        grid=(n // BLK,),
        in_specs=[pl.BlockSpec((BLK, HID), lambda i: (i, 0)),
                  pl.BlockSpec((8, HID), lambda i: (0, 0)),
                  pl.BlockSpec((1, HID), lambda i: (0, 0)),
                  pl.BlockSpec((1, HID), lambda i: (0, 0))],
        out_specs=pl.BlockSpec((BLK, HID), lambda i: (i, 0)),
        out_shape=jax.ShapeDtypeStruct((n, HID), jnp.float32),
    )(U, stats, g.reshape(1, HID), b.reshape(1, HID))


def _cls_body(g_ref, b1_ref, w2_ref, b2_ref, o_ref):
    gcat = jnp.concatenate([g_ref[0], g_ref[1]], axis=1)
    h = jnp.maximum(gcat + b1_ref[...], 0.0)
    o_ref[...] = (jnp.dot(h, w2_ref[...], preferred_element_type=jnp.float32)
                  + b2_ref[...])


def _classifier(G2, b1, W2, b2):
    return pl.pallas_call(
        _cls_body,
        grid=(NL // BLK,),
        in_specs=[pl.BlockSpec((2, BLK, 32), lambda i: (0, i, 0)),
                  pl.BlockSpec((1, HID), lambda i: (0, 0)),
                  pl.BlockSpec((HID, 1), lambda i: (0, 0)),
                  pl.BlockSpec((1, 1), lambda i: (0, 0))],
        out_specs=pl.BlockSpec((BLK, 1), lambda i: (i, 0)),
        out_shape=jax.ShapeDtypeStruct((NL, 1), jnp.float32),
    )(G2, b1.reshape(1, HID), W2.reshape(HID, 1), b2.reshape(1, 1))


def _agg_and_deg(x_user, x_movie, edge_src, edge_dst):
    aggu = jax.ops.segment_sum(jnp.take(x_movie, edge_dst, axis=0),
                               edge_src, num_segments=NU)
    aggm = jax.ops.segment_sum(jnp.take(x_user, edge_src, axis=0),
                               edge_dst, num_segments=NM)
    aggu2 = aggu.reshape(NU, 2, 32).transpose(1, 0, 2)
    aggm2 = aggm.reshape(NM, 2, 32).transpose(1, 0, 2)
    return aggu2, aggm2


def _label_gather(XU1, XM1, label_src, label_dst):
    G = jnp.take(XU1, label_src, axis=0) + jnp.take(XM1, label_dst, axis=0)
    return G.reshape(NL, 2, 32).transpose(1, 0, 2)


def kernel(movie_x, params, user_node_id, movie_node_id,
           edge_src, edge_dst, label_src, label_dst):
    p = params
    x_user = p['user_emb']
    x_movie = _movie_init(movie_x, p['movie_lin_W'], p['movie_lin_b'],
                          p['movie_emb'])
    ones = jnp.ones((NE,), jnp.float32)
    deg_u = jax.ops.segment_sum(ones, edge_src,
                                num_segments=NU).reshape(NU, 1)
    deg_m = jax.ops.segment_sum(ones, edge_dst,
                                num_segments=NM).reshape(NM, 1)
    for layer in range(2):
        aggu2, aggm2 = _agg_and_deg(x_user, x_movie, edge_src, edge_dst)
        U, ustats = _pass1(aggu2, deg_u, x_user,
                           p['sage%d_m2u_Wl' % layer],
                           p['sage%d_m2u_bl' % layer],
                           p['sage%d_m2u_Wr' % layer])
        M, mstats = _pass1(aggm2, deg_m, x_movie,
                           p['sage%d_u2m_Wl' % layer],
                           p['sage%d_u2m_bl' % layer],
                           p['sage%d_u2m_Wr' % layer])
        x_user = _pass2(U, ustats, p['bn%d_user_g' % layer],
                        p['bn%d_user_b' % layer])
        x_movie = _pass2(M, mstats, p['bn%d_movie_g' % layer],
                         p['bn%d_movie_b' % layer])
    XU1 = _matmul(x_user, p['cls_W1'][:HID])
    XM1 = _matmul(x_movie, p['cls_W1'][HID:])
    G2 = _label_gather(XU1, XM1, label_src, label_dst)
    out = _classifier(G2, p['cls_b1'], p['cls_W2'], p['cls_b2'])
    return out.reshape(-1)


# speedup vs baseline: 2.9436x; 2.9436x over previous
import functools

import jax
import jax.numpy as jnp
from jax import lax
from jax.experimental import pallas as pl
from jax.experimental.pallas import tpu as pltpu
from jax.experimental.pallas import tpu_sc as plsc

NU = 50000
NM = 10000
NE = 800000
NL = 100000
HID = 64
MF = 24
BLK = 1000

NSUB = 16
ECH = 80
NCHT = NE // NSUB // ECH
LCH = 80
NLCH = NL // LCH


def _mi_body(x_ref, w_ref, b_ref, emb_ref, o_ref):
    o_ref[...] = (jnp.dot(x_ref[...], w_ref[...],
                          preferred_element_type=jnp.float32)
                  + b_ref[...] + emb_ref[...])


def _movie_init(movie_x, W, b, emb):
    n = movie_x.shape[0]
    return pl.pallas_call(
        _mi_body,
        grid=(n // BLK,),
        in_specs=[pl.BlockSpec((BLK, MF), lambda i: (i, 0)),
                  pl.BlockSpec((MF, HID), lambda i: (0, 0)),
                  pl.BlockSpec((1, HID), lambda i: (0, 0)),
                  pl.BlockSpec((BLK, HID), lambda i: (i, 0))],
        out_specs=pl.BlockSpec((BLK, HID), lambda i: (i, 0)),
        out_shape=jax.ShapeDtypeStruct((n, HID), jnp.float32),
    )(movie_x, W, b.reshape(1, HID), emb)


def _mm_body(x_ref, w_ref, o_ref):
    o_ref[...] = jnp.dot(x_ref[...], w_ref[...],
                         preferred_element_type=jnp.float32)


def _matmul(x, W):
    n, k = x.shape
    m = W.shape[1]
    return pl.pallas_call(
        _mm_body,
        grid=(n // BLK,),
        in_specs=[pl.BlockSpec((BLK, k), lambda i: (i, 0)),
                  pl.BlockSpec((k, m), lambda i: (0, 0))],
        out_specs=pl.BlockSpec((BLK, m), lambda i: (i, 0)),
        out_shape=jax.ShapeDtypeStruct((n, m), jnp.float32),
    )(x, W)


def _p1_body(agg_ref, deg_ref, x_ref, wl_ref, bl_ref, wr_ref,
             u_ref, stats_ref, acc):
    i = pl.program_id(0)
    agg = jnp.concatenate([agg_ref[0], agg_ref[1]], axis=1)
    mean = agg * (1.0 / jnp.maximum(deg_ref[:, :1], 1.0))
    u = (jnp.dot(mean, wl_ref[...], preferred_element_type=jnp.float32)
         + bl_ref[...]
         + jnp.dot(x_ref[...], wr_ref[...],
                   preferred_element_type=jnp.float32))
    u_ref[...] = u

    @pl.when(i == 0)
    def _():
        acc[...] = jnp.zeros_like(acc)

    acc[0:1, :] += jnp.sum(u, axis=0, keepdims=True)
    acc[1:2, :] += jnp.sum(u * u, axis=0, keepdims=True)

    @pl.when(i == pl.num_programs(0) - 1)
    def _():
        n = BLK * pl.num_programs(0) * 1.0
        m = acc[0:1, :] / n
        v = acc[1:2, :] / n - m * m
        stats_ref[0:1, :] = m
        stats_ref[1:2, :] = 1.0 / jnp.sqrt(v + 1e-5)
        stats_ref[2:8, :] = jnp.zeros((6, HID), jnp.float32)


def _pass1(agg2, deg, x, Wl, bl, Wr):
    n = x.shape[0]
    return pl.pallas_call(
        _p1_body,
        grid=(n // BLK,),
        in_specs=[pl.BlockSpec((2, BLK, 32), lambda i: (0, i, 0)),
                  pl.BlockSpec((BLK, 16), lambda i: (i, 0)),
                  pl.BlockSpec((BLK, HID), lambda i: (i, 0)),
                  pl.BlockSpec((HID, HID), lambda i: (0, 0)),
                  pl.BlockSpec((1, HID), lambda i: (0, 0)),
                  pl.BlockSpec((HID, HID), lambda i: (0, 0))],
        out_specs=[pl.BlockSpec((BLK, HID), lambda i: (i, 0)),
                   pl.BlockSpec((8, HID), lambda i: (0, 0))],
        out_shape=[jax.ShapeDtypeStruct((n, HID), jnp.float32),
                   jax.ShapeDtypeStruct((8, HID), jnp.float32)],
        scratch_shapes=[pltpu.VMEM((8, HID), jnp.float32)],
    )(agg2, deg, x, Wl, bl.reshape(1, HID), Wr)


def _p2_body(u_ref, stats_ref, g_ref, b_ref, o_ref):
    o_ref[...] = jnp.maximum(
        (u_ref[...] - stats_ref[0:1, :]) * stats_ref[1:2, :] * g_ref[...]
        + b_ref[...], 0.0)


def _pass2(U, stats, g, b):
    n = U.shape[0]
    return pl.pallas_call(
        _p2_body,
        grid=(n // BLK,),
        in_specs=[pl.BlockSpec((BLK, HID), lambda i: (i, 0)),
                  pl.BlockSpec((8, HID), lambda i: (0, 0)),
                  pl.BlockSpec((1, HID), lambda i: (0, 0)),
                  pl.BlockSpec((1, HID), lambda i: (0, 0))],
        out_specs=pl.BlockSpec((BLK, HID), lambda i: (i, 0)),
        out_shape=jax.ShapeDtypeStruct((n, HID), jnp.float32),
    )(U, stats, g.reshape(1, HID), b.reshape(1, HID))


def _cls_body(g_ref, b1_ref, w2_ref, b2_ref, o_ref):
    h = jnp.maximum(g_ref[...] + b1_ref[...], 0.0)
    o_ref[...] = (jnp.dot(h, w2_ref[...], preferred_element_type=jnp.float32)
                  + b2_ref[...])


def _classifier(G, b1, W2, b2):
    return pl.pallas_call(
        _cls_body,
        grid=(NL // BLK,),
        in_specs=[pl.BlockSpec((BLK, HID), lambda i: (i, 0)),
                  pl.BlockSpec((1, HID), lambda i: (0, 0)),
                  pl.BlockSpec((HID, 1), lambda i: (0, 0)),
                  pl.BlockSpec((1, 1), lambda i: (0, 0))],
        out_specs=pl.BlockSpec((BLK, 1), lambda i: (i, 0)),
        out_shape=jax.ShapeDtypeStruct((NL, 1), jnp.float32),
    )(G, b1.reshape(1, HID), W2.reshape(HID, 1), b2.reshape(1, 1))


def _zero_chunks(s, arr_s, nchunks, zsrc):
    def zb(j, _):
        idx = s + j * NSUB

        @pl.when(idx < nchunks)
        def _():
            pltpu.sync_copy(zsrc, arr_s.at[pl.ds(idx * BLK, BLK)])
        return 0
    lax.fori_loop(0, (nchunks + NSUB - 1) // NSUB, zb, 0)


def _write_chunks(s, arr_s, out, out_base, nchunks):
    def wb(j, _):
        idx = s + j * NSUB

        @pl.when(idx < nchunks)
        def _():
            pltpu.sync_copy(arr_s.at[pl.ds(idx * BLK, BLK)],
                            out.at[pl.ds(out_base + idx * BLK, BLK)])
        return 0
    lax.fori_loop(0, (nchunks + NSUB - 1) // NSUB, wb, 0)


def _make_sc_agg_u():
    scratch = [pltpu.VMEM((ECH,), jnp.int32),
               pltpu.VMEM((ECH,), jnp.int32),
               pltpu.VMEM((ECH,), jnp.int32),
               pltpu.VMEM((ECH, 32), jnp.float32),
               pltpu.SemaphoreType.DMA,
               pltpu.VMEM_SHARED((NU, 32), jnp.float32)]
    mesh = plsc.VectorSubcoreMesh(core_axis_name="c", subcore_axis_name="s")

    def body(xm2, esrc, edst, z32, aggu_o,
             esrc_v, edst_v, gdst_v, rows_m, sem1, aggu_s):
        c = lax.axis_index("c")
        s = lax.axis_index("s")
        _zero_chunks(s, aggu_s, NU // BLK, z32)
        plsc.subcore_barrier()

        def chunk(k, _):
            base = (s * NCHT + k) * ECH
            pltpu.sync_copy(esrc.at[pl.ds(base, ECH)], esrc_v)
            pltpu.sync_copy(edst.at[pl.ds(base, ECH)], edst_v)
            for j in range(ECH // 16):
                gdst_v[pl.ds(j * 16, 16)] = edst_v[pl.ds(j * 16, 16)] * 2 + c
            pltpu.async_copy(xm2.at[gdst_v], rows_m, sem1).wait()
            pltpu.sync_copy(rows_m, aggu_s.at[esrc_v], add=True)
            return 0
        lax.fori_loop(0, NCHT, chunk, 0)
        plsc.subcore_barrier()
        _write_chunks(s, aggu_s, aggu_o, c * NU, NU // BLK)

    return functools.partial(
        pl.kernel, body,
        out_type=jax.ShapeDtypeStruct((2 * NU, 32), jnp.float32),
        mesh=mesh, scratch_types=scratch,
        compiler_params=pltpu.CompilerParams(use_tc_tiling_on_sc=False))()


def _make_sc_agg_m(with_deg):
    out_type = [jax.ShapeDtypeStruct((2 * NM, 32), jnp.float32)]
    scratch = [pltpu.VMEM((ECH,), jnp.int32),
               pltpu.VMEM((ECH,), jnp.int32),
               pltpu.VMEM((ECH,), jnp.int32),
               pltpu.VMEM((ECH, 32), jnp.float32),
               pltpu.SemaphoreType.DMA,
               pltpu.VMEM_SHARED((NM, 32), jnp.float32)]
    if with_deg:
        out_type += [jax.ShapeDtypeStruct((NU, 16), jnp.float32),
                     jax.ShapeDtypeStruct((NM, 16), jnp.float32)]
        scratch += [pltpu.VMEM((ECH, 16), jnp.float32),
                    pltpu.VMEM_SHARED((NU, 16), jnp.float32),
                    pltpu.VMEM_SHARED((NM, 16), jnp.float32)]
    mesh = plsc.VectorSubcoreMesh(core_axis_name="c", subcore_axis_name="s")

    def body(*args):
        if with_deg:
            (xu2, esrc, edst, z32, z1, ones1,
             aggm_o, degu_o, degm_o,
             esrc_v, edst_v, gsrc_v, rows_u, sem1, aggm_s,
             ones_v, degu_s, degm_s) = args
        else:
            (xu2, esrc, edst, z32,
             aggm_o,
             esrc_v, edst_v, gsrc_v, rows_u, sem1, aggm_s) = args
        c = lax.axis_index("c")
        s = lax.axis_index("s")
        _zero_chunks(s, aggm_s, NM // BLK, z32)
        if with_deg:
            _zero_chunks(s, degu_s, NU // BLK, z1)
            _zero_chunks(s, degm_s, NM // BLK, z1)
            pltpu.sync_copy(ones1, ones_v)
        plsc.subcore_barrier()

        def chunk(k, _):
            base = (s * NCHT + k) * ECH
            pltpu.sync_copy(esrc.at[pl.ds(base, ECH)], esrc_v)
            pltpu.sync_copy(edst.at[pl.ds(base, ECH)], edst_v)
            for j in range(ECH // 16):
                gsrc_v[pl.ds(j * 16, 16)] = esrc_v[pl.ds(j * 16, 16)] * 2 + c
            pltpu.async_copy(xu2.at[gsrc_v], rows_u, sem1).wait()
            pltpu.sync_copy(rows_u, aggm_s.at[edst_v], add=True)
            if with_deg:
                pltpu.sync_copy(ones_v, degu_s.at[esrc_v], add=True)
                pltpu.sync_copy(ones_v, degm_s.at[edst_v], add=True)
            return 0
        lax.fori_loop(0, NCHT, chunk, 0)
        plsc.subcore_barrier()
        _write_chunks(s, aggm_s, aggm_o, c * NM, NM // BLK)
        if with_deg:
            @pl.when(c == 0)
            def _():
                _write_chunks(s, degu_s, degu_o, 0, NU // BLK)
                _write_chunks(s, degm_s, degm_o, 0, NM // BLK)

    return functools.partial(
        pl.kernel, body, out_type=out_type, mesh=mesh,
        scratch_types=scratch,
        compiler_params=pltpu.CompilerParams(use_tc_tiling_on_sc=False))()


def _agg_and_deg(x_user, x_movie, edge_src, edge_dst, with_deg):
    xm2 = x_movie.reshape(2 * NM, 32)
    xu2 = x_user.reshape(2 * NU, 32)
    z32 = jnp.zeros((BLK, 32), jnp.float32)
    aggu = _make_sc_agg_u()(xm2, edge_src, edge_dst, z32)
    if with_deg:
        z1 = jnp.zeros((BLK, 16), jnp.float32)
        ones1 = jnp.ones((ECH, 16), jnp.float32)
        aggm, degu, degm = _make_sc_agg_m(True)(
            xu2, edge_src, edge_dst, z32, z1, ones1)
        return (aggu.reshape(2, NU, 32), aggm.reshape(2, NM, 32),
                degu, degm)
    aggm, = _make_sc_agg_m(False)(xu2, edge_src, edge_dst, z32)
    return aggu.reshape(2, NU, 32), aggm.reshape(2, NM, 32)


def _make_sc_labels():
    mesh = plsc.VectorSubcoreMesh(core_axis_name="c", subcore_axis_name="s")
    nw = 2 * NSUB
    per_w = NLCH // nw
    extra = NLCH - per_w * nw

    def body(xu1, xm1, ls_hbm, ld_hbm, out, lsv, ldv, ru, rm, sem1, sem2):
        c = lax.axis_index("c")
        s = lax.axis_index("s")
        w = c * NSUB + s
        nch = per_w + jnp.where(w < extra, 1, 0)
        start = per_w * w + jnp.minimum(w, extra)

        def chunk(k, _):
            base = (start + k) * LCH
            pltpu.sync_copy(ls_hbm.at[pl.ds(base, LCH)], lsv)
            pltpu.sync_copy(ld_hbm.at[pl.ds(base, LCH)], ldv)
            cu = pltpu.async_copy(xu1.at[lsv], ru, sem1)
            cm = pltpu.async_copy(xm1.at[ldv], rm, sem2)
            cu.wait()
            cm.wait()
            for r in range(LCH):
                for q in range(HID // 16):
                    ru[r, pl.ds(q * 16, 16)] += rm[r, pl.ds(q * 16, 16)]
            pltpu.sync_copy(ru, out.at[pl.ds(base, LCH)])
            return 0
        lax.fori_loop(0, nch, chunk, 0)

    return functools.partial(
        pl.kernel, body,
        out_type=jax.ShapeDtypeStruct((NL, HID), jnp.float32),
        mesh=mesh,
        scratch_types=[pltpu.VMEM((LCH,), jnp.int32),
                       pltpu.VMEM((LCH,), jnp.int32),
                       pltpu.VMEM((LCH, HID), jnp.float32),
                       pltpu.VMEM((LCH, HID), jnp.float32),
                       pltpu.SemaphoreType.DMA,
                       pltpu.SemaphoreType.DMA],
        compiler_params=pltpu.CompilerParams(use_tc_tiling_on_sc=False))()


def _label_gather(XU1, XM1, label_src, label_dst):
    return _make_sc_labels()(XU1, XM1, label_src, label_dst)


def kernel(movie_x, params, user_node_id, movie_node_id,
           edge_src, edge_dst, label_src, label_dst):
    p = params
    x_user = p['user_emb']
    x_movie = _movie_init(movie_x, p['movie_lin_W'], p['movie_lin_b'],
                          p['movie_emb'])
    deg_u = deg_m = None
    for layer in range(2):
        if layer == 0:
            aggu2, aggm2, deg_u, deg_m = _agg_and_deg(
                x_user, x_movie, edge_src, edge_dst, True)
        else:
            aggu2, aggm2 = _agg_and_deg(
                x_user, x_movie, edge_src, edge_dst, False)
        U, ustats = _pass1(aggu2, deg_u, x_user,
                           p['sage%d_m2u_Wl' % layer],
                           p['sage%d_m2u_bl' % layer],
                           p['sage%d_m2u_Wr' % layer])
        M, mstats = _pass1(aggm2, deg_m, x_movie,
                           p['sage%d_u2m_Wl' % layer],
                           p['sage%d_u2m_bl' % layer],
                           p['sage%d_u2m_Wr' % layer])
        x_user = _pass2(U, ustats, p['bn%d_user_g' % layer],
                        p['bn%d_user_b' % layer])
        x_movie = _pass2(M, mstats, p['bn%d_movie_g' % layer],
                         p['bn%d_movie_b' % layer])
    XU1 = _matmul(x_user, p['cls_W1'][:HID])
    XM1 = _matmul(x_movie, p['cls_W1'][HID:])
    G = _label_gather(XU1, XM1, label_src, label_dst)
    out = _classifier(G, p['cls_b1'], p['cls_W2'], p['cls_b2'])
    return out.reshape(-1)


# speedup vs baseline: 4.3687x; 1.4841x over previous
import functools

import jax
import jax.numpy as jnp
from jax import lax
from jax.experimental import pallas as pl
from jax.experimental.pallas import tpu as pltpu
from jax.experimental.pallas import tpu_sc as plsc

NU = 50000
NM = 10000
NE = 800000
NL = 100000
HID = 64
MF = 24
BLK = 1000

NSUB = 16
ECH = 80
NCHT = NE // NSUB // ECH
LCH = 80
NLCH = NL // LCH


def _mi_body(x_ref, w_ref, b_ref, emb_ref, o_ref):
    o_ref[...] = (jnp.dot(x_ref[...], w_ref[...],
                          preferred_element_type=jnp.float32)
                  + b_ref[...] + emb_ref[...])


def _movie_init(movie_x, W, b, emb):
    n = movie_x.shape[0]
    return pl.pallas_call(
        _mi_body,
        grid=(n // BLK,),
        in_specs=[pl.BlockSpec((BLK, MF), lambda i: (i, 0)),
                  pl.BlockSpec((MF, HID), lambda i: (0, 0)),
                  pl.BlockSpec((1, HID), lambda i: (0, 0)),
                  pl.BlockSpec((BLK, HID), lambda i: (i, 0))],
        out_specs=pl.BlockSpec((BLK, HID), lambda i: (i, 0)),
        out_shape=jax.ShapeDtypeStruct((n, HID), jnp.float32),
    )(movie_x, W, b.reshape(1, HID), emb)


def _mm_body(x_ref, w_ref, o_ref):
    o_ref[...] = jnp.dot(x_ref[...], w_ref[...],
                         preferred_element_type=jnp.float32)


def _matmul(x, W):
    n, k = x.shape
    m = W.shape[1]
    return pl.pallas_call(
        _mm_body,
        grid=(n // BLK,),
        in_specs=[pl.BlockSpec((BLK, k), lambda i: (i, 0)),
                  pl.BlockSpec((k, m), lambda i: (0, 0))],
        out_specs=pl.BlockSpec((BLK, m), lambda i: (i, 0)),
        out_shape=jax.ShapeDtypeStruct((n, m), jnp.float32),
    )(x, W)


def _p1_body(agg_ref, deg_ref, x_ref, wl_ref, bl_ref, wr_ref,
             u_ref, stats_ref, acc):
    i = pl.program_id(0)
    agg = jnp.concatenate([agg_ref[0], agg_ref[1]], axis=1)
    mean = agg * (1.0 / jnp.maximum(deg_ref[:, :1], 1.0))
    u = (jnp.dot(mean, wl_ref[...], preferred_element_type=jnp.float32)
         + bl_ref[...]
         + jnp.dot(x_ref[...], wr_ref[...],
                   preferred_element_type=jnp.float32))
    u_ref[...] = u

    @pl.when(i == 0)
    def _():
        acc[...] = jnp.zeros_like(acc)

    acc[0:1, :] += jnp.sum(u, axis=0, keepdims=True)
    acc[1:2, :] += jnp.sum(u * u, axis=0, keepdims=True)

    @pl.when(i == pl.num_programs(0) - 1)
    def _():
        n = BLK * pl.num_programs(0) * 1.0
        m = acc[0:1, :] / n
        v = acc[1:2, :] / n - m * m
        stats_ref[0:1, :] = m
        stats_ref[1:2, :] = 1.0 / jnp.sqrt(v + 1e-5)
        stats_ref[2:8, :] = jnp.zeros((6, HID), jnp.float32)


def _pass1(agg2, deg, x, Wl, bl, Wr):
    n = x.shape[0]
    return pl.pallas_call(
        _p1_body,
        grid=(n // BLK,),
        in_specs=[pl.BlockSpec((2, BLK, 32), lambda i: (0, i, 0)),
                  pl.BlockSpec((BLK, 16), lambda i: (i, 0)),
                  pl.BlockSpec((BLK, HID), lambda i: (i, 0)),
                  pl.BlockSpec((HID, HID), lambda i: (0, 0)),
                  pl.BlockSpec((1, HID), lambda i: (0, 0)),
                  pl.BlockSpec((HID, HID), lambda i: (0, 0))],
        out_specs=[pl.BlockSpec((BLK, HID), lambda i: (i, 0)),
                   pl.BlockSpec((8, HID), lambda i: (0, 0))],
        out_shape=[jax.ShapeDtypeStruct((n, HID), jnp.float32),
                   jax.ShapeDtypeStruct((8, HID), jnp.float32)],
        scratch_shapes=[pltpu.VMEM((8, HID), jnp.float32)],
    )(agg2, deg, x, Wl, bl.reshape(1, HID), Wr)


def _p2_body(u_ref, stats_ref, g_ref, b_ref, o_ref):
    o_ref[...] = jnp.maximum(
        (u_ref[...] - stats_ref[0:1, :]) * stats_ref[1:2, :] * g_ref[...]
        + b_ref[...], 0.0)


def _pass2(U, stats, g, b):
    n = U.shape[0]
    return pl.pallas_call(
        _p2_body,
        grid=(n // BLK,),
        in_specs=[pl.BlockSpec((BLK, HID), lambda i: (i, 0)),
                  pl.BlockSpec((8, HID), lambda i: (0, 0)),
                  pl.BlockSpec((1, HID), lambda i: (0, 0)),
                  pl.BlockSpec((1, HID), lambda i: (0, 0))],
        out_specs=pl.BlockSpec((BLK, HID), lambda i: (i, 0)),
        out_shape=jax.ShapeDtypeStruct((n, HID), jnp.float32),
    )(U, stats, g.reshape(1, HID), b.reshape(1, HID))


def _cls_body(g_ref, b1_ref, w2_ref, b2_ref, o_ref):
    h = jnp.maximum(g_ref[...] + b1_ref[...], 0.0)
    o_ref[...] = (jnp.dot(h, w2_ref[...], preferred_element_type=jnp.float32)
                  + b2_ref[...])


def _classifier(G, b1, W2, b2):
    return pl.pallas_call(
        _cls_body,
        grid=(NL // BLK,),
        in_specs=[pl.BlockSpec((BLK, HID), lambda i: (i, 0)),
                  pl.BlockSpec((1, HID), lambda i: (0, 0)),
                  pl.BlockSpec((HID, 1), lambda i: (0, 0)),
                  pl.BlockSpec((1, 1), lambda i: (0, 0))],
        out_specs=pl.BlockSpec((BLK, 1), lambda i: (i, 0)),
        out_shape=jax.ShapeDtypeStruct((NL, 1), jnp.float32),
    )(G, b1.reshape(1, HID), W2.reshape(HID, 1), b2.reshape(1, 1))


def _zero_chunks(s, arr_s, nchunks, zsrc):
    def zb(j, _):
        idx = s + j * NSUB

        @pl.when(idx < nchunks)
        def _():
            pltpu.sync_copy(zsrc, arr_s.at[pl.ds(idx * BLK, BLK)])
        return 0
    lax.fori_loop(0, (nchunks + NSUB - 1) // NSUB, zb, 0)


def _write_chunks(s, arr_s, out, out_base, nchunks):
    def wb(j, _):
        idx = s + j * NSUB

        @pl.when(idx < nchunks)
        def _():
            pltpu.sync_copy(arr_s.at[pl.ds(idx * BLK, BLK)],
                            out.at[pl.ds(out_base + idx * BLK, BLK)])
        return 0
    lax.fori_loop(0, (nchunks + NSUB - 1) // NSUB, wb, 0)


def _make_sc_agg_u():
    scratch = [pltpu.VMEM((2, ECH), jnp.int32),
               pltpu.VMEM((2, ECH), jnp.int32),
               pltpu.VMEM((2, ECH), jnp.int32),
               pltpu.VMEM((ECH, 32), jnp.float32),
               pltpu.VMEM((ECH, 32), jnp.float32),
               pltpu.SemaphoreType.DMA,
               pltpu.SemaphoreType.DMA,
               pltpu.VMEM_SHARED((NU, 32), jnp.float32)]
    mesh = plsc.VectorSubcoreMesh(core_axis_name="c", subcore_axis_name="s")

    def body(xm2, esrc, edst, z32, aggu_o,
             esrc_v, edst_v, gdst_v, rows0, rows1, sem0, sem1, aggu_s):
        c = lax.axis_index("c")
        s = lax.axis_index("s")
        rows = (rows0, rows1)
        sems = (sem0, sem1)
        _zero_chunks(s, aggu_s, NU // BLK, z32)

        def fire(k, b):
            base = (s * NCHT + k) * ECH
            pltpu.sync_copy(esrc.at[pl.ds(base, ECH)], esrc_v.at[b])
            pltpu.sync_copy(edst.at[pl.ds(base, ECH)], edst_v.at[b])
            for j in range(ECH // 16):
                gdst_v[b, pl.ds(j * 16, 16)] = (
                    edst_v[b, pl.ds(j * 16, 16)] * 2 + c)
            return pltpu.async_copy(xm2.at[gdst_v.at[b]], rows[b], sems[b])

        plsc.subcore_barrier()
        fire(0, 0)

        def pair(t, _):
            for b in range(2):
                k = 2 * t + b
                nb = 1 - b

                @pl.when(k < NCHT)
                def _():
                    @pl.when(k + 1 < NCHT)
                    def _():
                        fire(k + 1, nb)
                    pltpu.make_async_copy(xm2.at[gdst_v.at[b]], rows[b],
                                          sems[b]).wait()
                    pltpu.sync_copy(rows[b], aggu_s.at[esrc_v.at[b]],
                                    add=True)
            return 0
        lax.fori_loop(0, (NCHT + 1) // 2, pair, 0)
        plsc.subcore_barrier()
        _write_chunks(s, aggu_s, aggu_o, c * NU, NU // BLK)

    return functools.partial(
        pl.kernel, body,
        out_type=jax.ShapeDtypeStruct((2 * NU, 32), jnp.float32),
        mesh=mesh, scratch_types=scratch,
        compiler_params=pltpu.CompilerParams(use_tc_tiling_on_sc=False))()


def _make_sc_agg_m(with_deg):
    out_type = [jax.ShapeDtypeStruct((2 * NM, 32), jnp.float32)]
    scratch = [pltpu.VMEM((2, ECH), jnp.int32),
               pltpu.VMEM((2, ECH), jnp.int32),
               pltpu.VMEM((2, ECH), jnp.int32),
               pltpu.VMEM((ECH, 32), jnp.float32),
               pltpu.VMEM((ECH, 32), jnp.float32),
               pltpu.SemaphoreType.DMA,
               pltpu.SemaphoreType.DMA,
               pltpu.VMEM_SHARED((NM, 32), jnp.float32)]
    if with_deg:
        out_type += [jax.ShapeDtypeStruct((NU, 16), jnp.float32),
                     jax.ShapeDtypeStruct((NM, 16), jnp.float32)]
        scratch += [pltpu.VMEM((ECH, 16), jnp.float32),
                    pltpu.VMEM_SHARED((NU, 16), jnp.float32),
                    pltpu.VMEM_SHARED((NM, 16), jnp.float32)]
    mesh = plsc.VectorSubcoreMesh(core_axis_name="c", subcore_axis_name="s")

    def body(*args):
        if with_deg:
            (xu2, esrc, edst, z32, z1, ones1,
             aggm_o, degu_o, degm_o,
             esrc_v, edst_v, gsrc_v, rows0, rows1, sem0, sem1, aggm_s,
             ones_v, degu_s, degm_s) = args
        else:
            (xu2, esrc, edst, z32,
             aggm_o,
             esrc_v, edst_v, gsrc_v, rows0, rows1, sem0, sem1,
             aggm_s) = args
        c = lax.axis_index("c")
        s = lax.axis_index("s")
        rows = (rows0, rows1)
        sems = (sem0, sem1)
        _zero_chunks(s, aggm_s, NM // BLK, z32)
        if with_deg:
            _zero_chunks(s, degu_s, NU // BLK, z1)
            _zero_chunks(s, degm_s, NM // BLK, z1)
            pltpu.sync_copy(ones1, ones_v)
        plsc.subcore_barrier()

        def fire(k, b):
            base = (s * NCHT + k) * ECH
            pltpu.sync_copy(esrc.at[pl.ds(base, ECH)], esrc_v.at[b])
            pltpu.sync_copy(edst.at[pl.ds(base, ECH)], edst_v.at[b])
            for j in range(ECH // 16):
                gsrc_v[b, pl.ds(j * 16, 16)] = (
                    esrc_v[b, pl.ds(j * 16, 16)] * 2 + c)
            return pltpu.async_copy(xu2.at[gsrc_v.at[b]], rows[b], sems[b])

        fire(0, 0)

        def pair(t, _):
            for b in range(2):
                k = 2 * t + b
                nb = 1 - b

                @pl.when(k < NCHT)
                def _():
                    @pl.when(k + 1 < NCHT)
                    def _():
                        fire(k + 1, nb)
                    pltpu.make_async_copy(xu2.at[gsrc_v.at[b]], rows[b],
                                          sems[b]).wait()
                    pltpu.sync_copy(rows[b], aggm_s.at[edst_v.at[b]],
                                    add=True)
                    if with_deg:
                        pltpu.sync_copy(ones_v, degu_s.at[esrc_v.at[b]],
                                        add=True)
                        pltpu.sync_copy(ones_v, degm_s.at[edst_v.at[b]],
                                        add=True)
            return 0
        lax.fori_loop(0, (NCHT + 1) // 2, pair, 0)
        plsc.subcore_barrier()
        _write_chunks(s, aggm_s, aggm_o, c * NM, NM // BLK)
        if with_deg:
            @pl.when(c == 0)
            def _():
                _write_chunks(s, degu_s, degu_o, 0, NU // BLK)
                _write_chunks(s, degm_s, degm_o, 0, NM // BLK)

    return functools.partial(
        pl.kernel, body, out_type=out_type, mesh=mesh,
        scratch_types=scratch,
        compiler_params=pltpu.CompilerParams(use_tc_tiling_on_sc=False))()


def _agg_and_deg(x_user, x_movie, edge_src, edge_dst, with_deg):
    xm2 = x_movie.reshape(2 * NM, 32)
    xu2 = x_user.reshape(2 * NU, 32)
    z32 = jnp.zeros((BLK, 32), jnp.float32)
    aggu = _make_sc_agg_u()(xm2, edge_src, edge_dst, z32)
    if with_deg:
        z1 = jnp.zeros((BLK, 16), jnp.float32)
        ones1 = jnp.ones((ECH, 16), jnp.float32)
        aggm, degu, degm = _make_sc_agg_m(True)(
            xu2, edge_src, edge_dst, z32, z1, ones1)
        return (aggu.reshape(2, NU, 32), aggm.reshape(2, NM, 32),
                degu, degm)
    aggm, = _make_sc_agg_m(False)(xu2, edge_src, edge_dst, z32)
    return aggu.reshape(2, NU, 32), aggm.reshape(2, NM, 32)


def _make_sc_labels():
    mesh = plsc.VectorSubcoreMesh(core_axis_name="c", subcore_axis_name="s")
    nw = 2 * NSUB
    per_w = NLCH // nw
    extra = NLCH - per_w * nw

    def body(xu1, xm1, ls_hbm, ld_hbm, out, lsv, ldv, ru, rm, sem1, sem2):
        c = lax.axis_index("c")
        s = lax.axis_index("s")
        w = c * NSUB + s
        nch = per_w + jnp.where(w < extra, 1, 0)
        start = per_w * w + jnp.minimum(w, extra)

        def chunk(k, _):
            base = (start + k) * LCH
            pltpu.sync_copy(ls_hbm.at[pl.ds(base, LCH)], lsv)
            pltpu.sync_copy(ld_hbm.at[pl.ds(base, LCH)], ldv)
            cu = pltpu.async_copy(xu1.at[lsv], ru, sem1)
            cm = pltpu.async_copy(xm1.at[ldv], rm, sem2)
            cu.wait()
            cm.wait()
            for r in range(LCH):
                for q in range(HID // 16):
                    ru[r, pl.ds(q * 16, 16)] += rm[r, pl.ds(q * 16, 16)]
            pltpu.sync_copy(ru, out.at[pl.ds(base, LCH)])
            return 0
        lax.fori_loop(0, nch, chunk, 0)

    return functools.partial(
        pl.kernel, body,
        out_type=jax.ShapeDtypeStruct((NL, HID), jnp.float32),
        mesh=mesh,
        scratch_types=[pltpu.VMEM((LCH,), jnp.int32),
                       pltpu.VMEM((LCH,), jnp.int32),
                       pltpu.VMEM((LCH, HID), jnp.float32),
                       pltpu.VMEM((LCH, HID), jnp.float32),
                       pltpu.SemaphoreType.DMA,
                       pltpu.SemaphoreType.DMA],
        compiler_params=pltpu.CompilerParams(use_tc_tiling_on_sc=False))()


def _label_gather(XU1, XM1, label_src, label_dst):
    return _make_sc_labels()(XU1, XM1, label_src, label_dst)


def kernel(movie_x, params, user_node_id, movie_node_id,
           edge_src, edge_dst, label_src, label_dst):
    p = params
    x_user = p['user_emb']
    x_movie = _movie_init(movie_x, p['movie_lin_W'], p['movie_lin_b'],
                          p['movie_emb'])
    deg_u = deg_m = None
    for layer in range(2):
        if layer == 0:
            aggu2, aggm2, deg_u, deg_m = _agg_and_deg(
                x_user, x_movie, edge_src, edge_dst, True)
        else:
            aggu2, aggm2 = _agg_and_deg(
                x_user, x_movie, edge_src, edge_dst, False)
        U, ustats = _pass1(aggu2, deg_u, x_user,
                           p['sage%d_m2u_Wl' % layer],
                           p['sage%d_m2u_bl' % layer],
                           p['sage%d_m2u_Wr' % layer])
        M, mstats = _pass1(aggm2, deg_m, x_movie,
                           p['sage%d_u2m_Wl' % layer],
                           p['sage%d_u2m_bl' % layer],
                           p['sage%d_u2m_Wr' % layer])
        x_user = _pass2(U, ustats, p['bn%d_user_g' % layer],
                        p['bn%d_user_b' % layer])
        x_movie = _pass2(M, mstats, p['bn%d_movie_g' % layer],
                         p['bn%d_movie_b' % layer])
    XU1 = _matmul(x_user, p['cls_W1'][:HID])
    XM1 = _matmul(x_movie, p['cls_W1'][HID:])
    G = _label_gather(XU1, XM1, label_src, label_dst)
    out = _classifier(G, p['cls_b1'], p['cls_W2'], p['cls_b2'])
    return out.reshape(-1)


# speedup vs baseline: 5.5877x; 1.2790x over previous
import functools

import jax
import jax.numpy as jnp
from jax import lax
from jax.experimental import pallas as pl
from jax.experimental.pallas import tpu as pltpu
from jax.experimental.pallas import tpu_sc as plsc

NU = 50000
NM = 10000
NE = 800000
NL = 100000
HID = 64
MF = 24
BLK = 1000

NSUB = 16
ECH = 128
NECH = NE // ECH
EPT = NECH // NSUB
EXT = NECH - EPT * NSUB
LCH = 80
NLCH = NL // LCH


def _mi_body(x_ref, w_ref, b_ref, emb_ref, o_ref):
    o_ref[...] = (jnp.dot(x_ref[...], w_ref[...],
                          preferred_element_type=jnp.float32)
                  + b_ref[...] + emb_ref[...])


def _movie_init(movie_x, W, b, emb):
    n = movie_x.shape[0]
    return pl.pallas_call(
        _mi_body,
        grid=(n // BLK,),
        in_specs=[pl.BlockSpec((BLK, MF), lambda i: (i, 0)),
                  pl.BlockSpec((MF, HID), lambda i: (0, 0)),
                  pl.BlockSpec((1, HID), lambda i: (0, 0)),
                  pl.BlockSpec((BLK, HID), lambda i: (i, 0))],
        out_specs=pl.BlockSpec((BLK, HID), lambda i: (i, 0)),
        out_shape=jax.ShapeDtypeStruct((n, HID), jnp.float32),
    )(movie_x, W, b.reshape(1, HID), emb)


def _mm_body(x_ref, w_ref, o_ref):
    o_ref[...] = jnp.dot(x_ref[...], w_ref[...],
                         preferred_element_type=jnp.float32)


def _matmul(x, W):
    n, k = x.shape
    m = W.shape[1]
    return pl.pallas_call(
        _mm_body,
        grid=(n // BLK,),
        in_specs=[pl.BlockSpec((BLK, k), lambda i: (i, 0)),
                  pl.BlockSpec((k, m), lambda i: (0, 0))],
        out_specs=pl.BlockSpec((BLK, m), lambda i: (i, 0)),
        out_shape=jax.ShapeDtypeStruct((n, m), jnp.float32),
    )(x, W)


def _p1_body(agg_ref, deg_ref, x_ref, wl_ref, bl_ref, wr_ref,
             u_ref, stats_ref, acc):
    i = pl.program_id(0)
    agg = jnp.concatenate([agg_ref[0], agg_ref[1]], axis=1)
    mean = agg * (1.0 / jnp.maximum(deg_ref[:, :1], 1.0))
    u = (jnp.dot(mean, wl_ref[...], preferred_element_type=jnp.float32)
         + bl_ref[...]
         + jnp.dot(x_ref[...], wr_ref[...],
                   preferred_element_type=jnp.float32))
    u_ref[...] = u

    @pl.when(i == 0)
    def _():
        acc[...] = jnp.zeros_like(acc)

    acc[0:1, :] += jnp.sum(u, axis=0, keepdims=True)
    acc[1:2, :] += jnp.sum(u * u, axis=0, keepdims=True)

    @pl.when(i == pl.num_programs(0) - 1)
    def _():
        n = BLK * pl.num_programs(0) * 1.0
        m = acc[0:1, :] / n
        v = acc[1:2, :] / n - m * m
        stats_ref[0:1, :] = m
        stats_ref[1:2, :] = 1.0 / jnp.sqrt(v + 1e-5)
        stats_ref[2:8, :] = jnp.zeros((6, HID), jnp.float32)


def _pass1(agg2, deg, x, Wl, bl, Wr):
    n = x.shape[0]
    return pl.pallas_call(
        _p1_body,
        grid=(n // BLK,),
        in_specs=[pl.BlockSpec((2, BLK, 32), lambda i: (0, i, 0)),
                  pl.BlockSpec((BLK, 16), lambda i: (i, 0)),
                  pl.BlockSpec((BLK, HID), lambda i: (i, 0)),
                  pl.BlockSpec((HID, HID), lambda i: (0, 0)),
                  pl.BlockSpec((1, HID), lambda i: (0, 0)),
                  pl.BlockSpec((HID, HID), lambda i: (0, 0))],
        out_specs=[pl.BlockSpec((BLK, HID), lambda i: (i, 0)),
                   pl.BlockSpec((8, HID), lambda i: (0, 0))],
        out_shape=[jax.ShapeDtypeStruct((n, HID), jnp.float32),
                   jax.ShapeDtypeStruct((8, HID), jnp.float32)],
        scratch_shapes=[pltpu.VMEM((8, HID), jnp.float32)],
    )(agg2, deg, x, Wl, bl.reshape(1, HID), Wr)


def _p2_body(u_ref, stats_ref, g_ref, b_ref, o_ref):
    o_ref[...] = jnp.maximum(
        (u_ref[...] - stats_ref[0:1, :]) * stats_ref[1:2, :] * g_ref[...]
        + b_ref[...], 0.0)


def _pass2(U, stats, g, b):
    n = U.shape[0]
    return pl.pallas_call(
        _p2_body,
        grid=(n // BLK,),
        in_specs=[pl.BlockSpec((BLK, HID), lambda i: (i, 0)),
                  pl.BlockSpec((8, HID), lambda i: (0, 0)),
                  pl.BlockSpec((1, HID), lambda i: (0, 0)),
                  pl.BlockSpec((1, HID), lambda i: (0, 0))],
        out_specs=pl.BlockSpec((BLK, HID), lambda i: (i, 0)),
        out_shape=jax.ShapeDtypeStruct((n, HID), jnp.float32),
    )(U, stats, g.reshape(1, HID), b.reshape(1, HID))


def _cls_body(g_ref, b1_ref, w2_ref, b2_ref, o_ref):
    h = jnp.maximum(g_ref[...] + b1_ref[...], 0.0)
    o_ref[...] = (jnp.dot(h, w2_ref[...], preferred_element_type=jnp.float32)
                  + b2_ref[...])


def _classifier(G, b1, W2, b2):
    return pl.pallas_call(
        _cls_body,
        grid=(NL // BLK,),
        in_specs=[pl.BlockSpec((BLK, HID), lambda i: (i, 0)),
                  pl.BlockSpec((1, HID), lambda i: (0, 0)),
                  pl.BlockSpec((HID, 1), lambda i: (0, 0)),
                  pl.BlockSpec((1, 1), lambda i: (0, 0))],
        out_specs=pl.BlockSpec((BLK, 1), lambda i: (i, 0)),
        out_shape=jax.ShapeDtypeStruct((NL, 1), jnp.float32),
    )(G, b1.reshape(1, HID), W2.reshape(HID, 1), b2.reshape(1, 1))


def _zero_chunks(s, arr_s, nchunks, zsrc):
    def zb(j, _):
        idx = s + j * NSUB

        @pl.when(idx < nchunks)
        def _():
            pltpu.sync_copy(zsrc, arr_s.at[pl.ds(idx * BLK, BLK)])
        return 0
    lax.fori_loop(0, (nchunks + NSUB - 1) // NSUB, zb, 0)


def _write_chunks(s, arr_s, out, out_base, nchunks):
    def wb(j, _):
        idx = s + j * NSUB

        @pl.when(idx < nchunks)
        def _():
            pltpu.sync_copy(arr_s.at[pl.ds(idx * BLK, BLK)],
                            out.at[pl.ds(out_base + idx * BLK, BLK)])
        return 0
    lax.fori_loop(0, (nchunks + NSUB - 1) // NSUB, wb, 0)


def _make_sc_agg_u():
    scratch = [pltpu.VMEM((2, ECH), jnp.int32),
               pltpu.VMEM((2, ECH), jnp.int32),
               pltpu.VMEM((2, ECH), jnp.int32),
               pltpu.VMEM((ECH, 32), jnp.float32),
               pltpu.VMEM((ECH, 32), jnp.float32),
               pltpu.SemaphoreType.DMA,
               pltpu.SemaphoreType.DMA,
               pltpu.VMEM_SHARED((NU, 32), jnp.float32)]
    mesh = plsc.VectorSubcoreMesh(core_axis_name="c", subcore_axis_name="s")

    def body(xm2, esrc, edst, z32, aggu_o,
             esrc_v, edst_v, gdst_v, rows0, rows1, sem0, sem1, aggu_s):
        c = lax.axis_index("c")
        s = lax.axis_index("s")
        rows = (rows0, rows1)
        sems = (sem0, sem1)
        nch = EPT + jnp.where(s < EXT, 1, 0)
        start = EPT * s + jnp.minimum(s, EXT)
        _zero_chunks(s, aggu_s, NU // BLK, z32)

        def fire(k, b):
            pltpu.sync_copy(esrc.at[start + k], esrc_v.at[b])
            pltpu.sync_copy(edst.at[start + k], edst_v.at[b])
            for j in range(ECH // 16):
                gdst_v[b, pl.ds(j * 16, 16)] = (
                    edst_v[b, pl.ds(j * 16, 16)] * 2 + c)
            return pltpu.async_copy(xm2.at[gdst_v.at[b]], rows[b], sems[b])

        plsc.subcore_barrier()
        fire(0, 0)

        def pair(t, _):
            for b in range(2):
                k = 2 * t + b
                nb = 1 - b

                @pl.when(k < nch)
                def _():
                    @pl.when(k + 1 < nch)
                    def _():
                        fire(k + 1, nb)
                    pltpu.make_async_copy(xm2.at[gdst_v.at[b]], rows[b],
                                          sems[b]).wait()
                    pltpu.sync_copy(rows[b], aggu_s.at[esrc_v.at[b]],
                                    add=True)
            return 0
        lax.fori_loop(0, (EPT + 2) // 2, pair, 0)
        plsc.subcore_barrier()
        _write_chunks(s, aggu_s, aggu_o, c * NU, NU // BLK)

    return functools.partial(
        pl.kernel, body,
        out_type=jax.ShapeDtypeStruct((2 * NU, 32), jnp.float32),
        mesh=mesh, scratch_types=scratch,
        compiler_params=pltpu.CompilerParams(use_tc_tiling_on_sc=False))()


def _make_sc_agg_m(with_deg):
    out_type = [jax.ShapeDtypeStruct((2 * NM, 32), jnp.float32)]
    scratch = [pltpu.VMEM((2, ECH), jnp.int32),
               pltpu.VMEM((2, ECH), jnp.int32),
               pltpu.VMEM((2, ECH), jnp.int32),
               pltpu.VMEM((ECH, 32), jnp.float32),
               pltpu.VMEM((ECH, 32), jnp.float32),
               pltpu.SemaphoreType.DMA,
               pltpu.SemaphoreType.DMA,
               pltpu.VMEM_SHARED((NM, 32), jnp.float32)]
    if with_deg:
        out_type += [jax.ShapeDtypeStruct((NU, 16), jnp.float32),
                     jax.ShapeDtypeStruct((NM, 16), jnp.float32)]
        scratch += [pltpu.VMEM((ECH, 16), jnp.float32),
                    pltpu.VMEM_SHARED((NU, 16), jnp.float32),
                    pltpu.VMEM_SHARED((NM, 16), jnp.float32)]
    mesh = plsc.VectorSubcoreMesh(core_axis_name="c", subcore_axis_name="s")

    def body(*args):
        if with_deg:
            (xu2, esrc, edst, z32, z1, ones1,
             aggm_o, degu_o, degm_o,
             esrc_v, edst_v, gsrc_v, rows0, rows1, sem0, sem1, aggm_s,
             ones_v, degu_s, degm_s) = args
        else:
            (xu2, esrc, edst, z32,
             aggm_o,
             esrc_v, edst_v, gsrc_v, rows0, rows1, sem0, sem1,
             aggm_s) = args
        c = lax.axis_index("c")
        s = lax.axis_index("s")
        rows = (rows0, rows1)
        sems = (sem0, sem1)
        nch = EPT + jnp.where(s < EXT, 1, 0)
        start = EPT * s + jnp.minimum(s, EXT)
        _zero_chunks(s, aggm_s, NM // BLK, z32)
        if with_deg:
            _zero_chunks(s, degu_s, NU // BLK, z1)
            _zero_chunks(s, degm_s, NM // BLK, z1)
            pltpu.sync_copy(ones1, ones_v)
        plsc.subcore_barrier()

        def fire(k, b):
            pltpu.sync_copy(esrc.at[start + k], esrc_v.at[b])
            pltpu.sync_copy(edst.at[start + k], edst_v.at[b])
            for j in range(ECH // 16):
                gsrc_v[b, pl.ds(j * 16, 16)] = (
                    esrc_v[b, pl.ds(j * 16, 16)] * 2 + c)
            return pltpu.async_copy(xu2.at[gsrc_v.at[b]], rows[b], sems[b])

        fire(0, 0)

        def pair(t, _):
            for b in range(2):
                k = 2 * t + b
                nb = 1 - b

                @pl.when(k < nch)
                def _():
                    @pl.when(k + 1 < nch)
                    def _():
                        fire(k + 1, nb)
                    pltpu.make_async_copy(xu2.at[gsrc_v.at[b]], rows[b],
                                          sems[b]).wait()
                    pltpu.sync_copy(rows[b], aggm_s.at[edst_v.at[b]],
                                    add=True)
                    if with_deg:
                        pltpu.sync_copy(ones_v, degu_s.at[esrc_v.at[b]],
                                        add=True)
                        pltpu.sync_copy(ones_v, degm_s.at[edst_v.at[b]],
                                        add=True)
            return 0
        lax.fori_loop(0, (EPT + 2) // 2, pair, 0)
        plsc.subcore_barrier()
        _write_chunks(s, aggm_s, aggm_o, c * NM, NM // BLK)
        if with_deg:
            @pl.when(c == 0)
            def _():
                _write_chunks(s, degu_s, degu_o, 0, NU // BLK)
                _write_chunks(s, degm_s, degm_o, 0, NM // BLK)

    return functools.partial(
        pl.kernel, body, out_type=out_type, mesh=mesh,
        scratch_types=scratch,
        compiler_params=pltpu.CompilerParams(use_tc_tiling_on_sc=False))()


def _agg_and_deg(x_user, x_movie, edge_src, edge_dst, with_deg):
    xm2 = x_movie.reshape(2 * NM, 32)
    xu2 = x_user.reshape(2 * NU, 32)
    edge_src = edge_src.reshape(NECH, ECH)
    edge_dst = edge_dst.reshape(NECH, ECH)
    z32 = jnp.zeros((BLK, 32), jnp.float32)
    aggu = _make_sc_agg_u()(xm2, edge_src, edge_dst, z32)
    if with_deg:
        z1 = jnp.zeros((BLK, 16), jnp.float32)
        ones1 = jnp.ones((ECH, 16), jnp.float32)
        aggm, degu, degm = _make_sc_agg_m(True)(
            xu2, edge_src, edge_dst, z32, z1, ones1)
        return (aggu.reshape(2, NU, 32), aggm.reshape(2, NM, 32),
                degu, degm)
    aggm, = _make_sc_agg_m(False)(xu2, edge_src, edge_dst, z32)
    return aggu.reshape(2, NU, 32), aggm.reshape(2, NM, 32)


def _make_sc_labels():
    mesh = plsc.VectorSubcoreMesh(core_axis_name="c", subcore_axis_name="s")
    nw = 2 * NSUB
    per_w = NLCH // nw
    extra = NLCH - per_w * nw

    def body(xu1, xm1, ls_hbm, ld_hbm, out, lsv, ldv, ru, rm, sem1, sem2):
        c = lax.axis_index("c")
        s = lax.axis_index("s")
        w = c * NSUB + s
        nch = per_w + jnp.where(w < extra, 1, 0)
        start = per_w * w + jnp.minimum(w, extra)

        def chunk(k, _):
            base = (start + k) * LCH
            pltpu.sync_copy(ls_hbm.at[pl.ds(base, LCH)], lsv)
            pltpu.sync_copy(ld_hbm.at[pl.ds(base, LCH)], ldv)
            cu = pltpu.async_copy(xu1.at[lsv], ru, sem1)
            cm = pltpu.async_copy(xm1.at[ldv], rm, sem2)
            cu.wait()
            cm.wait()
            for r in range(LCH):
                for q in range(HID // 16):
                    ru[r, pl.ds(q * 16, 16)] += rm[r, pl.ds(q * 16, 16)]
            pltpu.sync_copy(ru, out.at[pl.ds(base, LCH)])
            return 0
        lax.fori_loop(0, nch, chunk, 0)

    return functools.partial(
        pl.kernel, body,
        out_type=jax.ShapeDtypeStruct((NL, HID), jnp.float32),
        mesh=mesh,
        scratch_types=[pltpu.VMEM((LCH,), jnp.int32),
                       pltpu.VMEM((LCH,), jnp.int32),
                       pltpu.VMEM((LCH, HID), jnp.float32),
                       pltpu.VMEM((LCH, HID), jnp.float32),
                       pltpu.SemaphoreType.DMA,
                       pltpu.SemaphoreType.DMA],
        compiler_params=pltpu.CompilerParams(use_tc_tiling_on_sc=False))()


def _label_gather(XU1, XM1, label_src, label_dst):
    return _make_sc_labels()(XU1, XM1, label_src, label_dst)


def kernel(movie_x, params, user_node_id, movie_node_id,
           edge_src, edge_dst, label_src, label_dst):
    p = params
    x_user = p['user_emb']
    x_movie = _movie_init(movie_x, p['movie_lin_W'], p['movie_lin_b'],
                          p['movie_emb'])
    deg_u = deg_m = None
    for layer in range(2):
        if layer == 0:
            aggu2, aggm2, deg_u, deg_m = _agg_and_deg(
                x_user, x_movie, edge_src, edge_dst, True)
        else:
            aggu2, aggm2 = _agg_and_deg(
                x_user, x_movie, edge_src, edge_dst, False)
        U, ustats = _pass1(aggu2, deg_u, x_user,
                           p['sage%d_m2u_Wl' % layer],
                           p['sage%d_m2u_bl' % layer],
                           p['sage%d_m2u_Wr' % layer])
        M, mstats = _pass1(aggm2, deg_m, x_movie,
                           p['sage%d_u2m_Wl' % layer],
                           p['sage%d_u2m_bl' % layer],
                           p['sage%d_u2m_Wr' % layer])
        x_user = _pass2(U, ustats, p['bn%d_user_g' % layer],
                        p['bn%d_user_b' % layer])
        x_movie = _pass2(M, mstats, p['bn%d_movie_g' % layer],
                         p['bn%d_movie_b' % layer])
    XU1 = _matmul(x_user, p['cls_W1'][:HID])
    XM1 = _matmul(x_movie, p['cls_W1'][HID:])
    G = _label_gather(XU1, XM1, label_src, label_dst)
    out = _classifier(G, p['cls_b1'], p['cls_W2'], p['cls_b2'])
    return out.reshape(-1)
